# Initial kernel scaffold; baseline (speedup 1.0000x reference)
import jax
import jax.numpy as jnp
from jax.experimental import pallas as pl


def kernel(x, W0, W1, W2, W3, W4, W5, W6, W7, W8, W9, W10, W11):
    raise NotImplementedError("write your pallas kernel here")


# speedup vs baseline: 10.4986x; 10.4986x over previous
import functools

import jax
import jax.numpy as jnp
import numpy as np
from jax import lax
from jax.experimental import pallas as pl
from jax.experimental.pallas import tpu as pltpu
from jax.experimental.pallas import tpu_sc as plsc

FG_DIMS = [11, 6, 6, 6, 6, 2, 2, 11, 8, 8, 8, 2]
BASES = [int(b) for b in np.cumsum([0] + FG_DIMS)[:12]]
TOTAL_ROWS = int(sum(FG_DIMS))
T_ROWS = 80
EMB = 64
NC, NS, L = 2, 16, 16
NW = NC * NS
R = 400


def _fg_kernel(n_rows: int):
    rows_per = n_rows // NW
    nchunks = rows_per // R
    mesh = plsc.VectorSubcoreMesh(core_axis_name="c", subcore_axis_name="s")

    @functools.partial(
        pl.kernel,
        out_type=jax.ShapeDtypeStruct((n_rows * EMB,), jnp.float32),
        mesh=mesh,
        scratch_types=[
            pltpu.VMEM((T_ROWS * EMB,), jnp.float32),
            pltpu.VMEM((R * 12,), jnp.int32),
            pltpu.VMEM((R * EMB,), jnp.float32),
        ],
        compiler_params=pltpu.CompilerParams(needs_layout_passes=False),
    )
    def k(x_hbm, t_hbm, out_hbm, tv, xv, ov):
        wid = lax.axis_index("s") * NC + lax.axis_index("c")
        pltpu.sync_copy(t_hbm, tv)
        iota = lax.iota(jnp.int32, L)
        cols = [iota + c * L for c in range(EMB // L)]

        def chunk_body(j, carry):
            base = wid * rows_per + j * R
            pltpu.sync_copy(x_hbm.at[pl.ds(base * 12, R * 12)], xv)

            def row_body(r, carry2):
                r12 = jnp.full((L,), 0, jnp.int32) + r * 12
                r64 = jnp.full((L,), 0, jnp.int32) + r * EMB
                acc = [jnp.zeros((L,), jnp.float32) for _ in range(EMB // L)]
                for i in range(12):
                    xval = plsc.load_gather(xv, [r12 + i])
                    flat = xval * EMB + BASES[i] * EMB
                    for c in range(EMB // L):
                        acc[c] = acc[c] + plsc.load_gather(tv, [flat + cols[c]])
                for c in range(EMB // L):
                    plsc.store_scatter(ov, [r64 + cols[c]], acc[c])
                return carry2

            lax.fori_loop(0, R, row_body, 0)
            pltpu.sync_copy(ov, out_hbm.at[pl.ds(base * EMB, R * EMB)])
            return carry

        lax.fori_loop(0, nchunks, chunk_body, 0)

    return k


def kernel(x, W0, W1, W2, W3, W4, W5, W6, W7, W8, W9, W10, W11):
    tables = [W0, W1, W2, W3, W4, W5, W6, W7, W8, W9, W10, W11]
    t = jnp.concatenate(tables, axis=0)
    t = jnp.pad(t, ((0, T_ROWS - TOTAL_ROWS), (0, 0))).reshape(-1)
    n = x.shape[0]
    x = x.astype(jnp.int32).reshape(-1)
    out = _fg_kernel(n)(x, t)
    return out.reshape(n, EMB)


# speedup vs baseline: 24.6942x; 2.3521x over previous
import functools

import jax
import jax.numpy as jnp
import numpy as np
from jax import lax
from jax.experimental import pallas as pl
from jax.experimental.pallas import tpu as pltpu
from jax.experimental.pallas import tpu_sc as plsc

FG_DIMS = [11, 6, 6, 6, 6, 2, 2, 11, 8, 8, 8, 2]
GROUPS = [(0, 7, 5), (8, 9, 10), (1, 2, 6), (3, 4, 11)]
GDIMS = [tuple(FG_DIMS[m] for m in g) for g in GROUPS]
GSIZES = [d0 * d1 * d2 for (d0, d1, d2) in GDIMS]
GBASES = [int(b) for b in np.cumsum([0] + GSIZES)[:4]]
T_ROWS = 904
EMB = 64
NC, NS, L = 2, 16, 16
NW = NC * NS
R = 400
NG = 4


def _fg_kernel(n_rows: int):
    rows_per = n_rows // NW
    nchunks = rows_per // R
    mesh = plsc.VectorSubcoreMesh(core_axis_name="c", subcore_axis_name="s")

    @functools.partial(
        pl.kernel,
        out_type=jax.ShapeDtypeStruct((n_rows * EMB,), jnp.float32),
        mesh=mesh,
        scratch_types=[
            pltpu.VMEM((T_ROWS * EMB,), jnp.float32),
            pltpu.VMEM((R * 12,), jnp.int32),
            pltpu.VMEM((NG * R,), jnp.int32),
            pltpu.VMEM((R * EMB,), jnp.float32),
        ],
        compiler_params=pltpu.CompilerParams(needs_layout_passes=False),
    )
    def k(x_hbm, t_hbm, out_hbm, tv, xv, iv, ov):
        wid = lax.axis_index("s") * NC + lax.axis_index("c")
        pltpu.sync_copy(t_hbm, tv)
        iota = lax.iota(jnp.int32, L)
        coffs = [iota + c * L for c in range(EMB // L)]
        lane12 = iota * 12

        def chunk_body(j, carry):
            base = wid * rows_per + j * R
            pltpu.sync_copy(x_hbm.at[pl.ds(base * 12, R * 12)], xv)

            @plsc.parallel_loop(0, R, step=L)
            def idx_body(r0):
                xbase = lane12 + r0 * 12
                for g, ((a, b, c), (_, db, dc)) in enumerate(zip(GROUPS, GDIMS)):
                    xa = plsc.load_gather(xv, [xbase + a])
                    xb = plsc.load_gather(xv, [xbase + b])
                    xc = plsc.load_gather(xv, [xbase + c])
                    idx = (xa * db + xb) * dc + xc
                    eidx = (idx + GBASES[g]) * EMB
                    plsc.store_scatter(iv, [iota + (g * R + r0)], eidx)

            @plsc.parallel_loop(0, R, step=1, unroll=2)
            def row_body(r):
                sp = [
                    plsc.load_gather(iv, [jnp.full((L,), g * R, jnp.int32) + r])
                    for g in range(NG)
                ]
                r64 = jnp.full((L,), 0, jnp.int32) + r * EMB
                for c in range(EMB // L):
                    v = [plsc.load_gather(tv, [sp[g] + coffs[c]]) for g in range(NG)]
                    acc = (v[0] + v[1]) + (v[2] + v[3])
                    plsc.store_scatter(ov, [r64 + coffs[c]], acc)

            pltpu.sync_copy(ov, out_hbm.at[pl.ds(base * EMB, R * EMB)])
            return carry

        lax.fori_loop(0, nchunks, chunk_body, 0)

    return k


def kernel(x, W0, W1, W2, W3, W4, W5, W6, W7, W8, W9, W10, W11):
    tables = [W0, W1, W2, W3, W4, W5, W6, W7, W8, W9, W10, W11]
    combined = []
    for a, b, c in GROUPS:
        t3 = (
            tables[a][:, None, None, :]
            + tables[b][None, :, None, :]
            + tables[c][None, None, :, :]
        )
        combined.append(t3.reshape(-1, EMB))
    t = jnp.concatenate(combined, axis=0)
    t = jnp.pad(t, ((0, T_ROWS - t.shape[0]), (0, 0))).reshape(-1)
    n = x.shape[0]
    x = x.astype(jnp.int32).reshape(-1)
    out = _fg_kernel(n)(x, t)
    return out.reshape(n, EMB)


# speedup vs baseline: 26.7130x; 1.0818x over previous
import functools

import jax
import jax.numpy as jnp
import numpy as np
from jax import lax
from jax.experimental import pallas as pl
from jax.experimental.pallas import tpu as pltpu
from jax.experimental.pallas import tpu_sc as plsc

FG_DIMS = [11, 6, 6, 6, 6, 2, 2, 11, 8, 8, 8, 2]
GROUPS = [(0, 7, 5), (8, 9, 10), (1, 2, 6), (3, 4, 11)]
GDIMS = [tuple(FG_DIMS[m] for m in g) for g in GROUPS]
GSIZES = [d0 * d1 * d2 for (d0, d1, d2) in GDIMS]
GBASES = [int(b) for b in np.cumsum([0] + GSIZES)[:4]]
T_ROWS = 904
EMB = 64
NC, NS, L = 2, 16, 16
NW = NC * NS
R = 400
NG = 4


def _fg_kernel(n_rows: int):
    rows_per = n_rows // NW
    nchunks = rows_per // R
    mesh = plsc.VectorSubcoreMesh(core_axis_name="c", subcore_axis_name="s")

    @functools.partial(
        pl.kernel,
        out_type=jax.ShapeDtypeStruct((n_rows * EMB,), jnp.float32),
        mesh=mesh,
        scratch_types=[
            pltpu.VMEM((T_ROWS * (EMB // 2),), jnp.int32),
            pltpu.VMEM((R * 12,), jnp.int32),
            pltpu.VMEM((NG * R,), jnp.int32),
            pltpu.VMEM((R * EMB,), jnp.float32),
        ],
        compiler_params=pltpu.CompilerParams(needs_layout_passes=False),
    )
    def k(x_hbm, t_hbm, out_hbm, tv, xv, iv, ov):
        wid = lax.axis_index("s") * NC + lax.axis_index("c")
        pltpu.sync_copy(t_hbm, tv)
        iota = lax.iota(jnp.int32, L)
        coffs = [iota + c * L for c in range(EMB // L)]

        lane12 = iota * 12

        def chunk_body(j, carry):
            base = wid * rows_per + j * R
            pltpu.sync_copy(x_hbm.at[pl.ds(base * 12, R * 12)], xv)

            @plsc.parallel_loop(0, R, step=L)
            def idx_body(r0):
                xbase = lane12 + r0 * 12
                for g, ((a, b, c), (_, db, dc)) in enumerate(zip(GROUPS, GDIMS)):
                    xa = plsc.load_gather(xv, [xbase + a])
                    xb = plsc.load_gather(xv, [xbase + b])
                    xc = plsc.load_gather(xv, [xbase + c])
                    idx = (xa * db + xb) * dc + xc
                    eidx = (idx + GBASES[g]) * (EMB // 2)
                    plsc.store_scatter(iv, [iota + (g * R + r0)], eidx)

            @plsc.parallel_loop(0, R, step=1, unroll=2)
            def row_body(r):
                sp = [
                    plsc.load_gather(iv, [jnp.full((L,), g * R, jnp.int32) + r])
                    for g in range(NG)
                ]
                r64 = jnp.full((L,), 0, jnp.int32) + r * EMB
                for h in range(2):
                    v = [
                        plsc.bitcast(
                            plsc.load_gather(tv, [sp[g] + coffs[h]]), jnp.bfloat16
                        )
                        for g in range(NG)
                    ]
                    acc = (v[0] + v[1]) + (v[2] + v[3])
                    au = plsc.bitcast(acc, jnp.int32)
                    lo = plsc.bitcast(au << 16, jnp.float32)
                    hi = plsc.bitcast(au & jnp.int32(-65536), jnp.float32)
                    plsc.store_scatter(ov, [r64 + coffs[h]], lo)
                    plsc.store_scatter(ov, [r64 + coffs[h + 2]], hi)

            pltpu.sync_copy(ov, out_hbm.at[pl.ds(base * EMB, R * EMB)])
            return carry

        lax.fori_loop(0, nchunks, chunk_body, 0)

    return k


def kernel(x, W0, W1, W2, W3, W4, W5, W6, W7, W8, W9, W10, W11):
    tables = [W0, W1, W2, W3, W4, W5, W6, W7, W8, W9, W10, W11]
    combined = []
    for a, b, c in GROUPS:
        t3 = (
            tables[a][:, None, None, :]
            + tables[b][None, :, None, :]
            + tables[c][None, None, :, :]
        )
        combined.append(t3.reshape(-1, EMB))
    t = jnp.concatenate(combined, axis=0)
    t = jnp.pad(t, ((0, T_ROWS - t.shape[0]), (0, 0)))
    tb = t.astype(jnp.bfloat16)
    lo = jax.lax.bitcast_convert_type(tb[:, : EMB // 2], jnp.uint16).astype(jnp.int32)
    hi = jax.lax.bitcast_convert_type(tb[:, EMB // 2 :], jnp.uint16).astype(jnp.int32)
    t = (lo | (hi << 16)).reshape(-1)
    n = x.shape[0]
    x = x.astype(jnp.int32).reshape(-1)
    out = _fg_kernel(n)(x, t)
    return out.reshape(n, EMB)


# speedup vs baseline: 33.9367x; 1.2704x over previous
import functools

import jax
import jax.numpy as jnp
import numpy as np
from jax import lax
from jax.experimental import pallas as pl
from jax.experimental.pallas import tpu as pltpu
from jax.experimental.pallas import tpu_sc as plsc

FG_DIMS = [11, 6, 6, 6, 6, 2, 2, 11, 8, 8, 8, 2]
GROUPS = [(0, 7, 5), (8, 9, 10), (1, 2, 6), (3, 4, 11)]
GDIMS = [tuple(FG_DIMS[m] for m in g) for g in GROUPS]
GSIZES = [d0 * d1 * d2 for (d0, d1, d2) in GDIMS]
GBASES = [int(b) for b in np.cumsum([0] + GSIZES)[:4]]
T_ROWS = 904
EMB = 64
NC, NS, L = 2, 16, 16
NW = NC * NS
R = 400
NG = 4


def _fg_kernel(n_rows: int):
    rows_per = n_rows // NW
    nchunks = rows_per // R
    mesh = plsc.VectorSubcoreMesh(core_axis_name="c", subcore_axis_name="s")

    @functools.partial(
        pl.kernel,
        out_type=jax.ShapeDtypeStruct((n_rows * EMB,), jnp.float32),
        mesh=mesh,
        scratch_types=[
            pltpu.VMEM((T_ROWS * (EMB // 2),), jnp.int32),
            pltpu.VMEM((R * 12,), jnp.int32),
            pltpu.VMEM((NG * R,), jnp.int32),
            pltpu.VMEM((R * EMB,), jnp.float32),
            pltpu.SemaphoreType.DMA,
        ],
        compiler_params=pltpu.CompilerParams(needs_layout_passes=False),
    )
    def k(x_hbm, t_hbm, out_hbm, tv, xv, iv, ov, sem):
        wid = lax.axis_index("s") * NC + lax.axis_index("c")
        pltpu.sync_copy(t_hbm, tv)
        iota = lax.iota(jnp.int32, L)
        coffs = [iota + c * L for c in range(EMB // L)]

        def chunk_body(j, carry):
            base = wid * rows_per + j * R
            descs = [
                pltpu.async_copy(
                    x_hbm.at[pl.ds(i * n_rows + base, R)],
                    xv.at[pl.ds(i * R, R)],
                    sem,
                )
                for i in range(12)
            ]
            for d in descs:
                d.wait()

            @plsc.parallel_loop(0, R, step=L)
            def idx_body(r0):
                for g, ((a, b, c), (_, db, dc)) in enumerate(zip(GROUPS, GDIMS)):
                    xa = xv[pl.ds(a * R + r0, L)]
                    xb = xv[pl.ds(b * R + r0, L)]
                    xc = xv[pl.ds(c * R + r0, L)]
                    idx = (xa * db + xb) * dc + xc
                    eidx = (idx + GBASES[g]) * (EMB // 2)
                    plsc.store_scatter(iv, [iota + (g * R + r0)], eidx)

            @plsc.parallel_loop(0, R, step=1, unroll=2)
            def row_body(r):
                sp = [
                    plsc.load_gather(iv, [jnp.full((L,), g * R, jnp.int32) + r])
                    for g in range(NG)
                ]
                r64 = jnp.full((L,), 0, jnp.int32) + r * EMB
                for h in range(2):
                    v = [
                        plsc.bitcast(
                            plsc.load_gather(tv, [sp[g] + coffs[h]]), jnp.bfloat16
                        )
                        for g in range(NG)
                    ]
                    acc = (v[0] + v[1]) + (v[2] + v[3])
                    au = plsc.bitcast(acc, jnp.int32)
                    lo = plsc.bitcast(au << 16, jnp.float32)
                    hi = plsc.bitcast(au & jnp.int32(-65536), jnp.float32)
                    plsc.store_scatter(ov, [r64 + coffs[h]], lo)
                    plsc.store_scatter(ov, [r64 + coffs[h + 2]], hi)

            pltpu.sync_copy(ov, out_hbm.at[pl.ds(base * EMB, R * EMB)])
            return carry

        lax.fori_loop(0, nchunks, chunk_body, 0)

    return k


def kernel(x, W0, W1, W2, W3, W4, W5, W6, W7, W8, W9, W10, W11):
    tables = [W0, W1, W2, W3, W4, W5, W6, W7, W8, W9, W10, W11]
    combined = []
    for a, b, c in GROUPS:
        t3 = (
            tables[a][:, None, None, :]
            + tables[b][None, :, None, :]
            + tables[c][None, None, :, :]
        )
        combined.append(t3.reshape(-1, EMB))
    t = jnp.concatenate(combined, axis=0)
    t = jnp.pad(t, ((0, T_ROWS - t.shape[0]), (0, 0)))
    tb = t.astype(jnp.bfloat16)
    lo = jax.lax.bitcast_convert_type(tb[:, : EMB // 2], jnp.uint16).astype(jnp.int32)
    hi = jax.lax.bitcast_convert_type(tb[:, EMB // 2 :], jnp.uint16).astype(jnp.int32)
    t = (lo | (hi << 16)).reshape(-1)
    n = x.shape[0]
    x = x.astype(jnp.int32)
    xcols = jnp.concatenate([x[:, i] for i in range(12)])
    out = _fg_kernel(n)(xcols, t)
    return out.reshape(n, EMB)
